# baseline (speedup 1.0000x reference)
import functools

import jax
import jax.numpy as jnp
from jax import lax
from jax.experimental import pallas as pl
from jax.experimental.pallas import tpu as pltpu
from jax.experimental.pallas import tpu_sc as plsc

_N = 500
_Q = 512
_NP = 12
_K4 = 4
_CF = 128
_NW = 32
_QT = _Q // _NW
_GT = _QT * _NP
_GH = _GT // 2
_INF = float("inf")


def _lrelu(x):
    return jnp.where(x >= 0, x, 0.01 * x)


def _mm(x, w):
    return lax.dot_general(x, w, (((1,), (1,)), ((), ())),
                           preferred_element_type=jnp.float32)


def _stage_a_body(imgA, imgB, ctAs, ctBs, ctAl, ctBl,
                  wc1, bc1, wc2, bc2, wp1, bp1, wp2, bp2,
                  fIA, fIB, fCA, fCB, idx_ab, w_ab, idx_bb, w_bb, idx_pp):
    def conv(x, w1, b1, w2, b2):
        return _mm(_lrelu(_mm(x, w1) + b1), w2) + b2

    fIA[...] = conv(imgA[...], wc1[...], bc1[...], wc2[...], bc2[...])
    fIB[...] = conv(imgB[...], wc1[...], bc1[...], wc2[...], bc2[...])
    fCA[...] = conv(ctAs[...], wp1[...], bp1[...], wp2[...], bp2[...])
    fCB[...] = conv(ctBs[...], wp1[...], bp1[...], wp2[...], bp2[...])

    jl = lax.broadcasted_iota(jnp.int32, (_Q, _Q), 1)
    i0 = lax.broadcasted_iota(jnp.int32, (_Q, _Q), 0)
    qrow_valid = lax.broadcasted_iota(jnp.int32, (_Q, 1), 0) < _N

    def dist2_qsub(qrys_sub, refs_lane):
        d = jnp.zeros((_Q, _Q), jnp.float32)
        for c in range(3):
            diff = qrys_sub[:, c:c + 1] - refs_lane[c:c + 1, :]
            d = d + diff * diff
        return jnp.where(jl >= _N, _INF, d)

    def topk_qsub(d, k):
        idxs, vals = [], []
        for j in range(k):
            mn = jnp.min(d, axis=1, keepdims=True)
            sel = jnp.min(jnp.where(d == mn, jl, _Q), axis=1,
                          keepdims=True)
            idxs.append(sel)
            vals.append(mn)
            d = jnp.where(jl == sel, _INF, d)
        return (jnp.concatenate(idxs, axis=1), jnp.concatenate(vals, axis=1))

    def emit(idx, v, idx_ref, w_ref):
        nd = -jnp.sqrt(jnp.maximum(v, 1e-12))
        m = jnp.max(jnp.where(qrow_valid, nd, -_INF))
        e = jnp.where(qrow_valid, jnp.exp(nd - m), 0.0)
        w = e / jnp.sum(e)
        zi = jnp.zeros((_Q, 16 - _NP), jnp.int32)
        zf = jnp.zeros((_Q, 16 - _NP), jnp.float32)
        idx_ref[...] = jnp.concatenate([idx, zi], axis=1)
        w_ref[...] = jnp.concatenate([w, zf], axis=1)

    emit(*topk_qsub(dist2_qsub(ctAs[...], ctBl[...]), _NP), idx_ab, w_ab)
    emit(*topk_qsub(dist2_qsub(ctAs[...], ctAl[...]), _NP), idx_bb, w_bb)

    d = jnp.zeros((_Q, _Q), jnp.float32)
    for c in range(3):
        diff = ctAs[...][:, c:c + 1] - ctBl[...][c:c + 1, :]
        d = d + diff * diff
    d = jnp.where(i0 >= _N, _INF, d)
    sels = []
    for j in range(_K4):
        mn = jnp.min(d, axis=0, keepdims=True)
        sel = jnp.min(jnp.where(d == mn, i0, _Q), axis=0, keepdims=True)
        sels.append(sel)
        d = jnp.where(i0 == sel, _INF, d)
    idx_pp[...] = jnp.concatenate(sels, axis=0)


def _stage_a(*args):
    f32, i32 = jnp.float32, jnp.int32
    outs = (
        jax.ShapeDtypeStruct((_Q, _CF), f32),
        jax.ShapeDtypeStruct((_Q, _CF), f32),
        jax.ShapeDtypeStruct((_Q, _CF), f32),
        jax.ShapeDtypeStruct((_Q, _CF), f32),
        jax.ShapeDtypeStruct((_Q, 16), i32),
        jax.ShapeDtypeStruct((_Q, 16), f32),
        jax.ShapeDtypeStruct((_Q, 16), i32),
        jax.ShapeDtypeStruct((_Q, 16), f32),
        jax.ShapeDtypeStruct((_K4, _Q), i32),
    )
    return pl.pallas_call(_stage_a_body, out_shape=outs)(*args)


def _make_sc(subtract):
    f32, i32 = jnp.float32, jnp.int32
    mesh = plsc.VectorSubcoreMesh(core_axis_name="c", subcore_axis_name="s",
                                  num_cores=2, num_subcores=16)
    nr = _QT * 16
    scratch = [
        pltpu.VMEM((4, nr // 4), i32),
        pltpu.VMEM((nr,), f32),
        pltpu.VMEM((nr, _CF), f32),
        pltpu.VMEM((nr, _CF), f32),
        pltpu.VMEM((_QT, _CF), f32),
        pltpu.VMEM((_QT, _CF), f32),
    ]
    if subtract:
        scratch += [pltpu.VMEM((_QT, _CF), f32), pltpu.VMEM((_QT, _CF), f32)]
    scratch.append(pltpu.SemaphoreType.DMA)
    out_type = (jax.ShapeDtypeStruct((_Q, _CF), f32),
                jax.ShapeDtypeStruct((_Q, _CF), f32))

    @functools.partial(pl.kernel, out_type=out_type, mesh=mesh,
                       scratch_types=scratch)
    def k(*refs):
        if subtract:
            (idx_hbm, w_hbm, tabI_hbm, tabC_hbm, ownI_hbm, ownC_hbm,
             outI, outC, idx_v, w_v, rI, rC, oI, oC, ownI_v, ownC_v,
             sem) = refs
        else:
            (idx_hbm, w_hbm, tabI_hbm, tabC_hbm,
             outI, outC, idx_v, w_v, rI, rC, oI, oC, sem) = refs
        nr = _QT * 16
        wid = lax.axis_index("s") * 2 + lax.axis_index("c")
        bq = wid * _QT
        pltpu.sync_copy(idx_hbm.at[pl.ds(wid * 4, 4)], idx_v)
        pltpu.sync_copy(w_hbm.at[pl.ds(wid * nr, nr)], w_v)
        cps = []
        for h in range(4):
            cps.append(pltpu.async_copy(
                tabI_hbm.at[idx_v.at[h]], rI.at[pl.ds(h * nr // 4, nr // 4)],
                sem))
            cps.append(pltpu.async_copy(
                tabC_hbm.at[idx_v.at[h]], rC.at[pl.ds(h * nr // 4, nr // 4)],
                sem))
        if subtract:
            pltpu.sync_copy(ownI_hbm.at[pl.ds(bq, _QT)], ownI_v)
            pltpu.sync_copy(ownC_hbm.at[pl.ds(bq, _QT)], ownC_v)
        for cp in cps:
            cp.wait()

        def body(q, carry):
            b0 = q * 16
            wq = w_v[pl.ds(b0, 16)]
            for ch in range(_CF // 16):
                sl = pl.ds(ch * 16, 16)
                mi = rI[b0, sl] * wq[0]
                mc = rC[b0, sl] * wq[0]
                for j in range(1, _NP):
                    mi = jnp.maximum(mi, rI[b0 + j, sl] * wq[j])
                    mc = jnp.maximum(mc, rC[b0 + j, sl] * wq[j])
                if subtract:
                    oI[q, sl] = ownI_v[q, sl] - mi
                    oC[q, sl] = ownC_v[q, sl] - mc
                else:
                    oI[q, sl] = mi
                    oC[q, sl] = mc
            return carry

        lax.fori_loop(0, _QT, body, 0)
        pltpu.sync_copy(oI, outI.at[pl.ds(bq, _QT)])
        pltpu.sync_copy(oC, outC.at[pl.ds(bq, _QT)])

    return k


def _stage_e_body(idiff, cdiff, s1, s2, tgt, cur, ipp,
                  wfc1, bfc1, wfc2, bfc2, wfu, bfu,
                  wp1, bp1, wp2, bp2, wp3, bp3, out):
    fi = _mm(jnp.concatenate([idiff[...], s1[...]], axis=1), wfc1[...]) + bfc1[...]
    fp = _mm(jnp.concatenate([cdiff[...], s2[...]], axis=1), wfc2[...]) + bfc2[...]
    ft = _mm(jnp.concatenate([fp, fi], axis=1), wfu[...]) + bfu[...]
    x = _mm(ft, wp1[...]) + bp1[...]
    x = _lrelu(_mm(x, wp2[...]) + bp2[...])
    x = _mm(x, wp3[...]) + bp3[...]
    tf = tgt[...] * x
    ii = lax.broadcasted_iota(jnp.int32, (_Q, _Q), 0)
    ippv = ipp[...]
    s = jnp.zeros((_Q, _Q), jnp.float32)
    for j in range(_K4):
        s = s + jnp.where(ii == ippv[j:j + 1, :], 1.0, 0.0)
    g = lax.dot_general(tf, s, (((0,), (0,)), ((), ())),
                        preferred_element_type=jnp.float32)
    out[...] = cur[...] + 0.25 * g[:, :_N]


def _stage_e(*args):
    return pl.pallas_call(
        _stage_e_body,
        out_shape=jax.ShapeDtypeStruct((160, _N), jnp.float32))(*args)


def kernel(img, cloud, img_tar, cloud_tar, current_feat, target_feat,
           w_conv1, b_conv1, w_conv2, b_conv2, w_pconv1, b_pconv1,
           w_pconv2, b_pconv2, w_fc1, b_fc1, w_fc2, b_fc2,
           w_fuse2, b_fuse2, w_pn1, b_pn1, w_pn2, b_pn2, w_pn3, b_pn3):
    padQ = lambda a: jnp.pad(a, ((0, _Q - _N), (0, 0)))
    row = lambda b: b[None, :]

    imgA = padQ(img_tar[0].T)
    imgB = padQ(img[0].T)
    ctA = cloud_tar[0]
    ctB = cloud[0]
    ctAs = jnp.pad(ctA, ((0, _Q - _N), (0, 5)))
    ctBs = jnp.pad(ctB, ((0, _Q - _N), (0, 5)))
    ctAl = jnp.pad(ctA.T, ((0, 5), (0, _Q - _N)))
    ctBl = jnp.pad(ctB.T, ((0, 5), (0, _Q - _N)))
    wp1p = jnp.pad(w_pconv1, ((0, 0), (0, 5)))

    (fIA, fIB, fCA, fCB, idx_ab, w_ab, idx_bb, w_bb, idx_pp) = _stage_a(
        imgA, imgB, ctAs, ctBs, ctAl, ctBl,
        w_conv1, row(b_conv1), w_conv2, row(b_conv2),
        wp1p, row(b_pconv1), w_pconv2, row(b_pconv2))

    idx_ab_f = idx_ab.reshape(_NW * 4, _QT * 4)
    idx_bb_f = idx_bb.reshape(_NW * 4, _QT * 4)
    w_ab_f = w_ab.reshape(-1)
    w_bb_f = w_bb.reshape(-1)

    idiff, cdiff = _make_sc(True)(idx_ab_f, w_ab_f, fIB, fCB, fIA, fCA)
    s2, s1 = _make_sc(False)(idx_bb_f, w_bb_f, idiff, cdiff)

    tgtT = padQ(target_feat[0].T)

    outp = _stage_e(idiff, cdiff, s1, s2, tgtT, current_feat[0], idx_pp,
                    w_fc1, row(b_fc1), w_fc2, row(b_fc2),
                    w_fuse2, row(b_fuse2), w_pn1, row(b_pn1),
                    w_pn2, row(b_pn2), w_pn3, row(b_pn3))
    return outp[None]


# speedup vs baseline: 3.2487x; 3.2487x over previous
import functools

import jax
import jax.numpy as jnp
from jax import lax
from jax.experimental import pallas as pl
from jax.experimental.pallas import tpu as pltpu
from jax.experimental.pallas import tpu_sc as plsc

_N = 500
_Q = 512
_NP = 12
_K4 = 4
_CF = 128
_NW = 32
_QT = _Q // _NW
_GT = _QT * _NP
_GH = _GT // 2
_INF = float("inf")


def _lrelu(x):
    return jnp.where(x >= 0, x, 0.01 * x)


def _mm(x, w):
    return lax.dot_general(x, w, (((1,), (1,)), ((), ())),
                           preferred_element_type=jnp.float32)


def _stage_a_body(imgA, imgB, ctAs, ctBs, ctAl, ctBl,
                  wc1, bc1, wc2, bc2, wp1, bp1, wp2, bp2,
                  fIA, fIB, fCA, fCB, idx_ab, w_ab, idx_bb, w_bb, idx_pp):
    def conv(x, w1, b1, w2, b2):
        return _mm(_lrelu(_mm(x, w1) + b1), w2) + b2

    fIA[...] = conv(imgA[...], wc1[...], bc1[...], wc2[...], bc2[...])
    fIB[...] = conv(imgB[...], wc1[...], bc1[...], wc2[...], bc2[...])
    fCA[...] = conv(ctAs[...], wp1[...], bp1[...], wp2[...], bp2[...])
    fCB[...] = conv(ctBs[...], wp1[...], bp1[...], wp2[...], bp2[...])

    jl = lax.broadcasted_iota(jnp.int32, (_Q, _Q), 1)
    i0 = lax.broadcasted_iota(jnp.int32, (_Q, _Q), 0)
    qrow_valid = lax.broadcasted_iota(jnp.int32, (_Q, 1), 0) < _N

    def dist2_qsub(qrys_sub, refs_lane):
        d = jnp.zeros((_Q, _Q), jnp.float32)
        for c in range(3):
            diff = qrys_sub[:, c:c + 1] - refs_lane[c:c + 1, :]
            d = d + diff * diff
        return jnp.where(jl >= _N, _INF, d)

    def topk_qsub(d, k):
        idxs, vals = [], []
        for j in range(k):
            mn = jnp.min(d, axis=1, keepdims=True)
            sel = jnp.min(jnp.where(d == mn, jl, _Q), axis=1,
                          keepdims=True)
            idxs.append(sel)
            vals.append(mn)
            d = jnp.where(jl == sel, _INF, d)
        return (jnp.concatenate(idxs, axis=1), jnp.concatenate(vals, axis=1))

    def emit(idx, v, idx_ref, w_ref):
        nd = -jnp.sqrt(jnp.maximum(v, 1e-12))
        m = jnp.max(jnp.where(qrow_valid, nd, -_INF))
        e = jnp.where(qrow_valid, jnp.exp(nd - m), 0.0)
        w = e / jnp.sum(e)
        zf = jnp.zeros((_Q, 16 - _NP), jnp.float32)
        idx_ref[...] = idx
        w_ref[...] = jnp.concatenate([w, zf], axis=1)

    emit(*topk_qsub(dist2_qsub(ctAs[...], ctBl[...]), _NP), idx_ab, w_ab)
    emit(*topk_qsub(dist2_qsub(ctAs[...], ctAl[...]), _NP), idx_bb, w_bb)

    d = jnp.zeros((_Q, _Q), jnp.float32)
    for c in range(3):
        diff = ctAs[...][:, c:c + 1] - ctBl[...][c:c + 1, :]
        d = d + diff * diff
    d = jnp.where(i0 >= _N, _INF, d)
    sels = []
    for j in range(_K4):
        mn = jnp.min(d, axis=0, keepdims=True)
        sel = jnp.min(jnp.where(d == mn, i0, _Q), axis=0, keepdims=True)
        sels.append(sel)
        d = jnp.where(i0 == sel, _INF, d)
    idx_pp[...] = jnp.concatenate(sels, axis=0)


def _stage_a(*args):
    f32, i32 = jnp.float32, jnp.int32
    outs = (
        jax.ShapeDtypeStruct((_Q, _CF), f32),
        jax.ShapeDtypeStruct((_Q, _CF), f32),
        jax.ShapeDtypeStruct((_Q, _CF), f32),
        jax.ShapeDtypeStruct((_Q, _CF), f32),
        jax.ShapeDtypeStruct((_Q, _NP), i32),
        jax.ShapeDtypeStruct((_Q, 16), f32),
        jax.ShapeDtypeStruct((_Q, _NP), i32),
        jax.ShapeDtypeStruct((_Q, 16), f32),
        jax.ShapeDtypeStruct((_K4, _Q), i32),
    )
    return pl.pallas_call(_stage_a_body, out_shape=outs)(*args)


def _make_sc(subtract):
    f32, i32 = jnp.float32, jnp.int32
    mesh = plsc.VectorSubcoreMesh(core_axis_name="c", subcore_axis_name="s",
                                  num_cores=2, num_subcores=16)
    scratch = [
        pltpu.VMEM((2, _GH), i32),
        pltpu.VMEM((_QT * 16,), f32),
        pltpu.VMEM((_GT, _CF), f32),
        pltpu.VMEM((_GT, _CF), f32),
        pltpu.VMEM((_QT, _CF), f32),
        pltpu.VMEM((_QT, _CF), f32),
    ]
    if subtract:
        scratch += [pltpu.VMEM((_QT, _CF), f32), pltpu.VMEM((_QT, _CF), f32)]
    scratch.append(pltpu.SemaphoreType.DMA)
    out_type = (jax.ShapeDtypeStruct((_Q, _CF), f32),
                jax.ShapeDtypeStruct((_Q, _CF), f32))

    @functools.partial(pl.kernel, out_type=out_type, mesh=mesh,
                       scratch_types=scratch)
    def k(*refs):
        if subtract:
            (idx_hbm, w_hbm, tabI_hbm, tabC_hbm, ownI_hbm, ownC_hbm,
             outI, outC, idx_v, w_v, rI, rC, oI, oC, ownI_v, ownC_v,
             sem) = refs
        else:
            (idx_hbm, w_hbm, tabI_hbm, tabC_hbm,
             outI, outC, idx_v, w_v, rI, rC, oI, oC, sem) = refs
        wid = lax.axis_index("s") * 2 + lax.axis_index("c")
        bq = wid * _QT
        pltpu.sync_copy(idx_hbm.at[pl.ds(wid * 2, 2)], idx_v)
        pltpu.sync_copy(w_hbm.at[pl.ds(wid * _QT * 16, _QT * 16)], w_v)
        cps = []
        for h in range(2):
            cps.append(pltpu.async_copy(
                tabI_hbm.at[idx_v.at[h]], rI.at[pl.ds(h * _GH, _GH)], sem))
            cps.append(pltpu.async_copy(
                tabC_hbm.at[idx_v.at[h]], rC.at[pl.ds(h * _GH, _GH)], sem))
        if subtract:
            pltpu.sync_copy(ownI_hbm.at[pl.ds(bq, _QT)], ownI_v)
            pltpu.sync_copy(ownC_hbm.at[pl.ds(bq, _QT)], ownC_v)
        for cp in cps:
            cp.wait()

        def body(q, carry):
            b0 = q * _NP
            wq = w_v[pl.ds(q * 16, 16)]
            for ch in range(_CF // 16):
                sl = pl.ds(ch * 16, 16)
                mi = rI[b0, sl] * wq[0]
                mc = rC[b0, sl] * wq[0]
                for j in range(1, _NP):
                    mi = jnp.maximum(mi, rI[b0 + j, sl] * wq[j])
                    mc = jnp.maximum(mc, rC[b0 + j, sl] * wq[j])
                if subtract:
                    oI[q, sl] = ownI_v[q, sl] - mi
                    oC[q, sl] = ownC_v[q, sl] - mc
                else:
                    oI[q, sl] = mi
                    oC[q, sl] = mc
            return carry

        lax.fori_loop(0, _QT, body, 0)
        pltpu.sync_copy(oI, outI.at[pl.ds(bq, _QT)])
        pltpu.sync_copy(oC, outC.at[pl.ds(bq, _QT)])

    return k


def _stage_e_body(idiff, cdiff, s1, s2, tgt, cur, ipp,
                  wfc1, bfc1, wfc2, bfc2, wfu, bfu,
                  wp1, bp1, wp2, bp2, wp3, bp3, out):
    fi = _mm(jnp.concatenate([idiff[...], s1[...]], axis=1), wfc1[...]) + bfc1[...]
    fp = _mm(jnp.concatenate([cdiff[...], s2[...]], axis=1), wfc2[...]) + bfc2[...]
    ft = _mm(jnp.concatenate([fp, fi], axis=1), wfu[...]) + bfu[...]
    x = _mm(ft, wp1[...]) + bp1[...]
    x = _lrelu(_mm(x, wp2[...]) + bp2[...])
    x = _mm(x, wp3[...]) + bp3[...]
    tf = tgt[...] * x
    ii = lax.broadcasted_iota(jnp.int32, (_Q, _Q), 0)
    ippv = ipp[...]
    s = jnp.zeros((_Q, _Q), jnp.float32)
    for j in range(_K4):
        s = s + jnp.where(ii == ippv[j:j + 1, :], 1.0, 0.0)
    g = lax.dot_general(tf, s, (((0,), (0,)), ((), ())),
                        preferred_element_type=jnp.float32)
    out[...] = cur[...] + 0.25 * g[:, :_N]


def _stage_e(*args):
    return pl.pallas_call(
        _stage_e_body,
        out_shape=jax.ShapeDtypeStruct((160, _N), jnp.float32))(*args)


def kernel(img, cloud, img_tar, cloud_tar, current_feat, target_feat,
           w_conv1, b_conv1, w_conv2, b_conv2, w_pconv1, b_pconv1,
           w_pconv2, b_pconv2, w_fc1, b_fc1, w_fc2, b_fc2,
           w_fuse2, b_fuse2, w_pn1, b_pn1, w_pn2, b_pn2, w_pn3, b_pn3):
    padQ = lambda a: jnp.pad(a, ((0, _Q - _N), (0, 0)))
    row = lambda b: b[None, :]

    imgA = padQ(img_tar[0].T)
    imgB = padQ(img[0].T)
    ctA = cloud_tar[0]
    ctB = cloud[0]
    ctAs = jnp.pad(ctA, ((0, _Q - _N), (0, 5)))
    ctBs = jnp.pad(ctB, ((0, _Q - _N), (0, 5)))
    ctAl = jnp.pad(ctA.T, ((0, 5), (0, _Q - _N)))
    ctBl = jnp.pad(ctB.T, ((0, 5), (0, _Q - _N)))
    wp1p = jnp.pad(w_pconv1, ((0, 0), (0, 5)))

    (fIA, fIB, fCA, fCB, idx_ab, w_ab, idx_bb, w_bb, idx_pp) = _stage_a(
        imgA, imgB, ctAs, ctBs, ctAl, ctBl,
        w_conv1, row(b_conv1), w_conv2, row(b_conv2),
        wp1p, row(b_pconv1), w_pconv2, row(b_pconv2))

    idx_ab_f = idx_ab.reshape(_NW * 2, _GH)
    idx_bb_f = idx_bb.reshape(_NW * 2, _GH)
    w_ab_f = w_ab.reshape(-1)
    w_bb_f = w_bb.reshape(-1)

    idiff, cdiff = _make_sc(True)(idx_ab_f, w_ab_f, fIB, fCB, fIA, fCA)
    s2, s1 = _make_sc(False)(idx_bb_f, w_bb_f, idiff, cdiff)

    tgtT = padQ(target_feat[0].T)

    outp = _stage_e(idiff, cdiff, s1, s2, tgtT, current_feat[0], idx_pp,
                    w_fc1, row(b_fc1), w_fc2, row(b_fc2),
                    w_fuse2, row(b_fuse2), w_pn1, row(b_pn1),
                    w_pn2, row(b_pn2), w_pn3, row(b_pn3))
    return outp[None]


# speedup vs baseline: 3.3855x; 1.0421x over previous
import functools

import jax
import jax.numpy as jnp
from jax import lax
from jax.experimental import pallas as pl
from jax.experimental.pallas import tpu as pltpu
from jax.experimental.pallas import tpu_sc as plsc

_N = 500
_Q = 512
_NP = 12
_K4 = 4
_CF = 128
_NW = 32
_QT = _Q // _NW
_GT = _QT * _NP
_GH = _GT // 2
_INF = float("inf")


def _lrelu(x):
    return jnp.where(x >= 0, x, 0.01 * x)


def _mm(x, w):
    return lax.dot_general(x, w, (((1,), (1,)), ((), ())),
                           preferred_element_type=jnp.float32)


def _stage_a_body(imgB_r, ctB_r, imgA_r, ctA_r,
                  wc1, bc1, wc2, bc2, wp1, bp1, wp2, bp2,
                  fIA, fIB, fCA, fCB, idx_ab, w_ab, idx_bb, w_bb, idx_pp):
    zpt = jnp.zeros((_Q - _N, 32), jnp.float32)
    zc = jnp.zeros((_Q - _N, 3), jnp.float32)
    imgA = jnp.concatenate([jnp.transpose(imgA_r[0]), zpt], axis=0)
    imgB = jnp.concatenate([jnp.transpose(imgB_r[0]), zpt], axis=0)
    ctAs = jnp.concatenate([ctA_r[0], zc], axis=0)
    ctBs = jnp.concatenate([ctB_r[0], zc], axis=0)
    ctAl = jnp.transpose(ctAs)
    ctBl = jnp.transpose(ctBs)

    def conv(x, w1, b1, w2, b2):
        return _mm(_lrelu(_mm(x, w1) + b1), w2) + b2

    fIA[...] = conv(imgA, wc1[...], bc1[...], wc2[...], bc2[...])
    fIB[...] = conv(imgB, wc1[...], bc1[...], wc2[...], bc2[...])
    fCA[...] = conv(ctAs, wp1[...], bp1[...], wp2[...], bp2[...])
    fCB[...] = conv(ctBs, wp1[...], bp1[...], wp2[...], bp2[...])

    jl = lax.broadcasted_iota(jnp.int32, (_Q, _Q), 1)
    i0 = lax.broadcasted_iota(jnp.int32, (_Q, _Q), 0)
    qrow_valid = lax.broadcasted_iota(jnp.int32, (_Q, 1), 0) < _N

    def dist2_qsub(qrys_sub, refs_lane):
        d = jnp.zeros((_Q, _Q), jnp.float32)
        for c in range(3):
            diff = qrys_sub[:, c:c + 1] - refs_lane[c:c + 1, :]
            d = d + diff * diff
        return jnp.where(jl >= _N, _INF, d)

    def topk_qsub(d, k):
        idxs, vals = [], []
        for j in range(k):
            mn = jnp.min(d, axis=1, keepdims=True)
            sel = jnp.min(jnp.where(d == mn, jl, _Q), axis=1,
                          keepdims=True)
            idxs.append(sel)
            vals.append(mn)
            d = jnp.where(jl == sel, _INF, d)
        return (jnp.concatenate(idxs, axis=1), jnp.concatenate(vals, axis=1))

    def emit(idx, v, idx_ref, w_ref):
        nd = -jnp.sqrt(jnp.maximum(v, 1e-12))
        m = jnp.max(jnp.where(qrow_valid, nd, -_INF))
        e = jnp.where(qrow_valid, jnp.exp(nd - m), 0.0)
        w = e / jnp.sum(e)
        zf = jnp.zeros((_Q, 16 - _NP), jnp.float32)
        idx_ref[...] = idx
        w_ref[...] = jnp.concatenate([w, zf], axis=1)

    emit(*topk_qsub(dist2_qsub(ctAs, ctBl), _NP), idx_ab, w_ab)
    emit(*topk_qsub(dist2_qsub(ctAs, ctAl), _NP), idx_bb, w_bb)

    d = jnp.zeros((_Q, _Q), jnp.float32)
    for c in range(3):
        diff = ctAs[:, c:c + 1] - ctBl[c:c + 1, :]
        d = d + diff * diff
    d = jnp.where(i0 >= _N, _INF, d)
    sels = []
    for j in range(_K4):
        mn = jnp.min(d, axis=0, keepdims=True)
        sel = jnp.min(jnp.where(d == mn, i0, _Q), axis=0, keepdims=True)
        sels.append(sel)
        d = jnp.where(i0 == sel, _INF, d)
    idx_pp[...] = jnp.concatenate(sels, axis=0)


def _stage_a(*args):
    f32, i32 = jnp.float32, jnp.int32
    outs = (
        jax.ShapeDtypeStruct((_Q, _CF), f32),
        jax.ShapeDtypeStruct((_Q, _CF), f32),
        jax.ShapeDtypeStruct((_Q, _CF), f32),
        jax.ShapeDtypeStruct((_Q, _CF), f32),
        jax.ShapeDtypeStruct((_Q, _NP), i32),
        jax.ShapeDtypeStruct((_Q, 16), f32),
        jax.ShapeDtypeStruct((_Q, _NP), i32),
        jax.ShapeDtypeStruct((_Q, 16), f32),
        jax.ShapeDtypeStruct((_K4, _Q), i32),
    )
    return pl.pallas_call(_stage_a_body, out_shape=outs)(*args)


def _make_sc(subtract):
    f32, i32 = jnp.float32, jnp.int32
    mesh = plsc.VectorSubcoreMesh(core_axis_name="c", subcore_axis_name="s",
                                  num_cores=2, num_subcores=16)
    scratch = [
        pltpu.VMEM((2, _GH), i32),
        pltpu.VMEM((_QT * 16,), f32),
        pltpu.VMEM((_GT, _CF), f32),
        pltpu.VMEM((_GT, _CF), f32),
        pltpu.VMEM((_QT, _CF), f32),
        pltpu.VMEM((_QT, _CF), f32),
    ]
    if subtract:
        scratch += [pltpu.VMEM((_QT, _CF), f32), pltpu.VMEM((_QT, _CF), f32)]
    scratch.append(pltpu.SemaphoreType.DMA)
    out_type = (jax.ShapeDtypeStruct((_Q, _CF), f32),
                jax.ShapeDtypeStruct((_Q, _CF), f32))

    @functools.partial(pl.kernel, out_type=out_type, mesh=mesh,
                       scratch_types=scratch)
    def k(*refs):
        if subtract:
            (idx_hbm, w_hbm, tabI_hbm, tabC_hbm, ownI_hbm, ownC_hbm,
             outI, outC, idx_v, w_v, rI, rC, oI, oC, ownI_v, ownC_v,
             sem) = refs
        else:
            (idx_hbm, w_hbm, tabI_hbm, tabC_hbm,
             outI, outC, idx_v, w_v, rI, rC, oI, oC, sem) = refs
        wid = lax.axis_index("s") * 2 + lax.axis_index("c")
        bq = wid * _QT
        pltpu.sync_copy(idx_hbm.at[pl.ds(wid * 2, 2)], idx_v)
        pltpu.sync_copy(w_hbm.at[pl.ds(wid * _QT * 16, _QT * 16)], w_v)
        cps = []
        for h in range(2):
            cps.append(pltpu.async_copy(
                tabI_hbm.at[idx_v.at[h]], rI.at[pl.ds(h * _GH, _GH)], sem))
            cps.append(pltpu.async_copy(
                tabC_hbm.at[idx_v.at[h]], rC.at[pl.ds(h * _GH, _GH)], sem))
        if subtract:
            pltpu.sync_copy(ownI_hbm.at[pl.ds(bq, _QT)], ownI_v)
            pltpu.sync_copy(ownC_hbm.at[pl.ds(bq, _QT)], ownC_v)
        for cp in cps:
            cp.wait()

        @plsc.parallel_loop(0, _QT, 1, unroll=2)
        def body(q):
            b0 = q * _NP
            wq = w_v[pl.ds(q * 16, 16)]
            for ch in range(_CF // 16):
                sl = pl.ds(ch * 16, 16)
                mi = rI[b0, sl] * wq[0]
                mc = rC[b0, sl] * wq[0]
                for j in range(1, _NP):
                    mi = jnp.maximum(mi, rI[b0 + j, sl] * wq[j])
                    mc = jnp.maximum(mc, rC[b0 + j, sl] * wq[j])
                if subtract:
                    oI[q, sl] = ownI_v[q, sl] - mi
                    oC[q, sl] = ownC_v[q, sl] - mc
                else:
                    oI[q, sl] = mi
                    oC[q, sl] = mc
        pltpu.sync_copy(oI, outI.at[pl.ds(bq, _QT)])
        pltpu.sync_copy(oC, outC.at[pl.ds(bq, _QT)])

    return k


def _stage_e_body(idiff, cdiff, s1, s2, tgt_r, cur_r, ipp,
                  wfc1, bfc1, wfc2, bfc2, wfu, bfu,
                  wp1, bp1, wp2, bp2, wp3, bp3, out):
    fi = _mm(jnp.concatenate([idiff[...], s1[...]], axis=1), wfc1[...]) + bfc1[...]
    fp = _mm(jnp.concatenate([cdiff[...], s2[...]], axis=1), wfc2[...]) + bfc2[...]
    ft = _mm(jnp.concatenate([fp, fi], axis=1), wfu[...]) + bfu[...]
    x = _mm(ft, wp1[...]) + bp1[...]
    x = _lrelu(_mm(x, wp2[...]) + bp2[...])
    x = _mm(x, wp3[...]) + bp3[...]
    xT = jnp.transpose(x)
    tf = jnp.concatenate(
        [tgt_r[0] * xT[:, :_N], jnp.zeros((160, _Q - _N), jnp.float32)],
        axis=1)
    ii = lax.broadcasted_iota(jnp.int32, (_Q, _Q), 0)
    ippv = ipp[...]
    s = jnp.zeros((_Q, _Q), jnp.float32)
    for j in range(_K4):
        s = s + jnp.where(ii == ippv[j:j + 1, :], 1.0, 0.0)
    g = lax.dot_general(tf, s, (((1,), (0,)), ((), ())),
                        preferred_element_type=jnp.float32)
    out[...] = cur_r[0] + 0.25 * g[:, :_N]


def _stage_e(*args):
    return pl.pallas_call(
        _stage_e_body,
        out_shape=jax.ShapeDtypeStruct((160, _N), jnp.float32))(*args)


def kernel(img, cloud, img_tar, cloud_tar, current_feat, target_feat,
           w_conv1, b_conv1, w_conv2, b_conv2, w_pconv1, b_pconv1,
           w_pconv2, b_pconv2, w_fc1, b_fc1, w_fc2, b_fc2,
           w_fuse2, b_fuse2, w_pn1, b_pn1, w_pn2, b_pn2, w_pn3, b_pn3):
    row = lambda b: b[None, :]

    (fIA, fIB, fCA, fCB, idx_ab, w_ab, idx_bb, w_bb, idx_pp) = _stage_a(
        img, cloud, img_tar, cloud_tar,
        w_conv1, row(b_conv1), w_conv2, row(b_conv2),
        w_pconv1, row(b_pconv1), w_pconv2, row(b_pconv2))

    idx_ab_f = idx_ab.reshape(_NW * 2, _GH)
    idx_bb_f = idx_bb.reshape(_NW * 2, _GH)
    w_ab_f = w_ab.reshape(-1)
    w_bb_f = w_bb.reshape(-1)

    idiff, cdiff = _make_sc(True)(idx_ab_f, w_ab_f, fIB, fCB, fIA, fCA)
    s2, s1 = _make_sc(False)(idx_bb_f, w_bb_f, idiff, cdiff)

    outp = _stage_e(idiff, cdiff, s1, s2, target_feat, current_feat, idx_pp,
                    w_fc1, row(b_fc1), w_fc2, row(b_fc2),
                    w_fuse2, row(b_fuse2), w_pn1, row(b_pn1),
                    w_pn2, row(b_pn2), w_pn3, row(b_pn3))
    return outp[None]


# speedup vs baseline: 3.7301x; 1.1018x over previous
import functools

import jax
import jax.numpy as jnp
from jax import lax
from jax.experimental import pallas as pl
from jax.experimental.pallas import tpu as pltpu
from jax.experimental.pallas import tpu_sc as plsc

_N = 500
_Q = 512
_NP = 12
_K4 = 4
_CF = 128
_NW = 32
_QT = _Q // _NW
_GT = _QT * _NP
_GH = _GT // 2
_INF = float("inf")


def _lrelu(x):
    return jnp.where(x >= 0, x, 0.01 * x)


def _mm(x, w):
    return lax.dot_general(x, w, (((1,), (1,)), ((), ())),
                           preferred_element_type=jnp.float32)


def _stage_a_body(imgB_r, ctB_r, imgA_r, ctA_r,
                  wc1, bc1, wc2, bc2, wp1, bp1, wp2, bp2,
                  fG, ownG, idx_ab, w_ab, idx_bb, w_bb, idx_pp):
    zpt = jnp.zeros((_Q - _N, 32), jnp.float32)
    zc = jnp.zeros((_Q - _N, 3), jnp.float32)
    imgA = jnp.concatenate([jnp.transpose(imgA_r[0]), zpt], axis=0)
    imgB = jnp.concatenate([jnp.transpose(imgB_r[0]), zpt], axis=0)
    ctAs = jnp.concatenate([ctA_r[0], zc], axis=0)
    ctBs = jnp.concatenate([ctB_r[0], zc], axis=0)
    ctAl = jnp.transpose(ctAs)
    ctBl = jnp.transpose(ctBs)

    def conv(x, w1, b1, w2, b2):
        return _mm(_lrelu(_mm(x, w1) + b1), w2) + b2

    ownG[0] = conv(imgA, wc1[...], bc1[...], wc2[...], bc2[...])
    fG[0] = conv(imgB, wc1[...], bc1[...], wc2[...], bc2[...])
    ownG[1] = conv(ctAs, wp1[...], bp1[...], wp2[...], bp2[...])
    fG[1] = conv(ctBs, wp1[...], bp1[...], wp2[...], bp2[...])

    jl = lax.broadcasted_iota(jnp.int32, (_Q, _Q), 1)
    i0 = lax.broadcasted_iota(jnp.int32, (_Q, _Q), 0)
    qrow_valid = lax.broadcasted_iota(jnp.int32, (_Q, 1), 0) < _N

    def dist2_qsub(qrys_sub, refs_lane):
        d = jnp.zeros((_Q, _Q), jnp.float32)
        for c in range(3):
            diff = qrys_sub[:, c:c + 1] - refs_lane[c:c + 1, :]
            d = d + diff * diff
        return jnp.where(jl >= _N, _INF, d)

    def topk_qsub(d, k):
        idxs, vals = [], []
        for j in range(k):
            mn = jnp.min(d, axis=1, keepdims=True)
            sel = jnp.min(jnp.where(d == mn, jl, _Q), axis=1,
                          keepdims=True)
            idxs.append(sel)
            vals.append(mn)
            d = jnp.where(jl == sel, _INF, d)
        return (jnp.concatenate(idxs, axis=1), jnp.concatenate(vals, axis=1))

    def emit(idx, v, idx_ref, w_ref):
        nd = -jnp.sqrt(jnp.maximum(v, 1e-12))
        m = jnp.max(jnp.where(qrow_valid, nd, -_INF))
        e = jnp.where(qrow_valid, jnp.exp(nd - m), 0.0)
        w = e / jnp.sum(e)
        zf = jnp.zeros((_Q, 16 - _NP), jnp.float32)
        idx_ref[...] = idx
        w_ref[...] = jnp.concatenate([w, zf], axis=1)

    emit(*topk_qsub(dist2_qsub(ctAs, ctBl), _NP), idx_ab, w_ab)
    emit(*topk_qsub(dist2_qsub(ctAs, ctAl), _NP), idx_bb, w_bb)

    d = jnp.zeros((_Q, _Q), jnp.float32)
    for c in range(3):
        diff = ctAs[:, c:c + 1] - ctBl[c:c + 1, :]
        d = d + diff * diff
    d = jnp.where(i0 >= _N, _INF, d)
    sels = []
    for j in range(_K4):
        mn = jnp.min(d, axis=0, keepdims=True)
        sel = jnp.min(jnp.where(d == mn, i0, _Q), axis=0, keepdims=True)
        sels.append(sel)
        d = jnp.where(i0 == sel, _INF, d)
    idx_pp[...] = jnp.concatenate(sels, axis=0)


def _stage_a(*args):
    f32, i32 = jnp.float32, jnp.int32
    outs = (
        jax.ShapeDtypeStruct((2, _Q, _CF), f32),
        jax.ShapeDtypeStruct((2, _Q, _CF), f32),
        jax.ShapeDtypeStruct((_Q, _NP), i32),
        jax.ShapeDtypeStruct((_Q, 16), f32),
        jax.ShapeDtypeStruct((_Q, _NP), i32),
        jax.ShapeDtypeStruct((_Q, 16), f32),
        jax.ShapeDtypeStruct((_K4, _Q), i32),
    )
    return pl.pallas_call(_stage_a_body, out_shape=outs)(*args)


_QS = _Q // 16
_GS = _QS * _NP


def _sc_pool():
    f32, i32 = jnp.float32, jnp.int32
    mesh = plsc.VectorSubcoreMesh(core_axis_name="c", subcore_axis_name="s",
                                  num_cores=2, num_subcores=16)
    scratch = [
        pltpu.VMEM((4, _GH), i32),
        pltpu.VMEM((_QS * 16,), f32),
        pltpu.VMEM((_GS, _CF), f32),
        pltpu.VMEM((_QS, _CF), f32),
        pltpu.VMEM((_QS, _CF), f32),
        pltpu.VMEM((_QS, _CF), f32),
        pltpu.VMEM_SHARED((_Q, _CF), f32),
        pltpu.SemaphoreType.DMA,
        pltpu.SemaphoreType.DMA,
    ]
    out_type = (jax.ShapeDtypeStruct((2, _Q, _CF), f32),
                jax.ShapeDtypeStruct((2, _Q, _CF), f32))

    @functools.partial(pl.kernel, out_type=out_type, mesh=mesh,
                       scratch_types=scratch)
    def k(idxa_hbm, wa_hbm, idxb_hbm, wb_hbm, tabG, ownG,
          diffG, sG, idx_v, w_v, rows, own_v, oD, oS, spm, sem, sem2):
        c = lax.axis_index("c")
        s = lax.axis_index("s")
        bq = s * _QS

        def pool(oref, sub):
            @plsc.parallel_loop(0, _QS, 1, unroll=2)
            def body(q):
                b0 = q * _NP
                wq = w_v[pl.ds(q * 16, 16)]
                for ch in range(_CF // 16):
                    sl = pl.ds(ch * 16, 16)
                    m = rows[b0, sl] * wq[0]
                    for j in range(1, _NP):
                        m = jnp.maximum(m, rows[b0 + j, sl] * wq[j])
                    if sub:
                        oref[q, sl] = own_v[q, sl] - m
                    else:
                        oref[q, sl] = m

        pltpu.sync_copy(idxa_hbm.at[pl.ds(s * 4, 4)], idx_v)
        pltpu.sync_copy(wa_hbm.at[pl.ds(s * _QS * 16, _QS * 16)], w_v)
        cps = [pltpu.async_copy(tabG.at[c].at[idx_v.at[h]],
                                rows.at[pl.ds(h * _GH, _GH)], sem)
               for h in range(4)]
        pltpu.sync_copy(ownG.at[c].at[pl.ds(bq, _QS)], own_v)
        for cp in cps:
            cp.wait()
        pool(oD, True)
        pltpu.sync_copy(oD, spm.at[pl.ds(bq, _QS)])
        hbm_cp = pltpu.async_copy(oD, diffG.at[c].at[pl.ds(bq, _QS)], sem2)
        plsc.subcore_barrier()

        pltpu.sync_copy(idxb_hbm.at[pl.ds(s * 4, 4)], idx_v)
        pltpu.sync_copy(wb_hbm.at[pl.ds(s * _QS * 16, _QS * 16)], w_v)
        cps = [pltpu.async_copy(spm.at[idx_v.at[h]],
                                rows.at[pl.ds(h * _GH, _GH)], sem)
               for h in range(4)]
        for cp in cps:
            cp.wait()
        pool(oS, False)
        pltpu.sync_copy(oS, sG.at[c].at[pl.ds(bq, _QS)])
        hbm_cp.wait()

    return k


def _stage_e_body(diffG, sG, tgt_r, cur_r, ipp,
                  wfc1, bfc1, wfc2, bfc2, wfu, bfu,
                  wp1, bp1, wp2, bp2, wp3, bp3, out):
    fi = _mm(jnp.concatenate([diffG[0], sG[1]], axis=1), wfc1[...]) + bfc1[...]
    fp = _mm(jnp.concatenate([diffG[1], sG[0]], axis=1), wfc2[...]) + bfc2[...]
    ft = _mm(jnp.concatenate([fp, fi], axis=1), wfu[...]) + bfu[...]
    x = _mm(ft, wp1[...]) + bp1[...]
    x = _lrelu(_mm(x, wp2[...]) + bp2[...])
    x = _mm(x, wp3[...]) + bp3[...]
    xT = jnp.transpose(x)
    tf = jnp.concatenate(
        [tgt_r[0] * xT[:, :_N], jnp.zeros((160, _Q - _N), jnp.float32)],
        axis=1)
    ii = lax.broadcasted_iota(jnp.int32, (_Q, _Q), 0)
    ippv = ipp[...]
    s = jnp.zeros((_Q, _Q), jnp.float32)
    for j in range(_K4):
        s = s + jnp.where(ii == ippv[j:j + 1, :], 1.0, 0.0)
    g = lax.dot_general(tf, s, (((1,), (0,)), ((), ())),
                        preferred_element_type=jnp.float32)
    out[...] = cur_r[0] + 0.25 * g[:, :_N]


def _stage_e(*args):
    return pl.pallas_call(
        _stage_e_body,
        out_shape=jax.ShapeDtypeStruct((160, _N), jnp.float32))(*args)


def kernel(img, cloud, img_tar, cloud_tar, current_feat, target_feat,
           w_conv1, b_conv1, w_conv2, b_conv2, w_pconv1, b_pconv1,
           w_pconv2, b_pconv2, w_fc1, b_fc1, w_fc2, b_fc2,
           w_fuse2, b_fuse2, w_pn1, b_pn1, w_pn2, b_pn2, w_pn3, b_pn3):
    row = lambda b: b[None, :]

    (fG, ownG, idx_ab, w_ab, idx_bb, w_bb, idx_pp) = _stage_a(
        img, cloud, img_tar, cloud_tar,
        w_conv1, row(b_conv1), w_conv2, row(b_conv2),
        w_pconv1, row(b_pconv1), w_pconv2, row(b_pconv2))

    idx_ab_f = idx_ab.reshape(-1, _GH)
    idx_bb_f = idx_bb.reshape(-1, _GH)
    w_ab_f = w_ab.reshape(-1)
    w_bb_f = w_bb.reshape(-1)

    diffG, sG = _sc_pool()(idx_ab_f, w_ab_f, idx_bb_f, w_bb_f, fG, ownG)

    outp = _stage_e(diffG, sG, target_feat, current_feat, idx_pp,
                    w_fc1, row(b_fc1), w_fc2, row(b_fc2),
                    w_fuse2, row(b_fuse2), w_pn1, row(b_pn1),
                    w_pn2, row(b_pn2), w_pn3, row(b_pn3))
    return outp[None]


# speedup vs baseline: 3.7967x; 1.0178x over previous
import functools

import jax
import jax.numpy as jnp
from jax import lax
from jax.experimental import pallas as pl
from jax.experimental.pallas import tpu as pltpu
from jax.experimental.pallas import tpu_sc as plsc

_N = 500
_Q = 512
_NP = 12
_K4 = 4
_CF = 128
_NW = 32
_QT = _Q // _NW
_GT = _QT * _NP
_GH = _GT // 2
_INF = float("inf")


def _lrelu(x):
    return jnp.where(x >= 0, x, 0.01 * x)


def _mm(x, w):
    return lax.dot_general(x, w, (((1,), (1,)), ((), ())),
                           preferred_element_type=jnp.float32)


def _stage_a_body(imgB_r, ctB_r, imgA_r, ctA_r,
                  wc1, bc1, wc2, bc2, wp1, bp1, wp2, bp2,
                  fG, ownG, idx_ab, w_ab, idx_bb, w_bb, idx_pp):
    zpt = jnp.zeros((_Q - _N, 32), jnp.float32)
    zc = jnp.zeros((_Q - _N, 3), jnp.float32)
    imgA = jnp.concatenate([jnp.transpose(imgA_r[0]), zpt], axis=0)
    imgB = jnp.concatenate([jnp.transpose(imgB_r[0]), zpt], axis=0)
    ctAs = jnp.concatenate([ctA_r[0], zc], axis=0)
    ctBs = jnp.concatenate([ctB_r[0], zc], axis=0)
    ctAl = jnp.transpose(ctAs)
    ctBl = jnp.transpose(ctBs)

    def conv(x, w1, b1, w2, b2):
        return _mm(_lrelu(_mm(x, w1) + b1), w2) + b2

    ownG[0] = conv(imgA, wc1[...], bc1[...], wc2[...], bc2[...])
    fG[0] = conv(imgB, wc1[...], bc1[...], wc2[...], bc2[...])
    ownG[1] = conv(ctAs, wp1[...], bp1[...], wp2[...], bp2[...])
    fG[1] = conv(ctBs, wp1[...], bp1[...], wp2[...], bp2[...])

    jl = lax.broadcasted_iota(jnp.int32, (_Q, _Q), 1)
    i0 = lax.broadcasted_iota(jnp.int32, (_Q, _Q), 0)
    qrow_valid = lax.broadcasted_iota(jnp.int32, (_Q, 1), 0) < _N

    def dist2_qsub(qrys_sub, refs_lane):
        d = jnp.zeros((_Q, _Q), jnp.float32)
        for c in range(3):
            diff = qrys_sub[:, c:c + 1] - refs_lane[c:c + 1, :]
            d = d + diff * diff
        return jnp.where(jl >= _N, _INF, d)

    def topk_qsub(d, k):
        idxs, vals = [], []
        for j in range(k):
            mn = jnp.min(d, axis=1, keepdims=True)
            sel = jnp.min(jnp.where(d == mn, jl, _Q), axis=1,
                          keepdims=True)
            idxs.append(sel)
            vals.append(mn)
            d = jnp.where(jl == sel, _INF, d)
        return (jnp.concatenate(idxs, axis=1), jnp.concatenate(vals, axis=1))

    def emit(idx, v, idx_ref, w_ref):
        nd = -jnp.sqrt(jnp.maximum(v, 1e-12))
        m = jnp.max(jnp.where(qrow_valid, nd, -_INF))
        e = jnp.where(qrow_valid, jnp.exp(nd - m), 0.0)
        w = e / jnp.sum(e)
        zf = jnp.zeros((_Q, 16 - _NP), jnp.float32)
        idx_ref[...] = idx
        w_ref[...] = jnp.concatenate([w, zf], axis=1)

    emit(*topk_qsub(dist2_qsub(ctAs, ctBl), _NP), idx_ab, w_ab)
    emit(*topk_qsub(dist2_qsub(ctAs, ctAl), _NP), idx_bb, w_bb)

    d = jnp.zeros((_Q, _Q), jnp.float32)
    for c in range(3):
        diff = ctAs[:, c:c + 1] - ctBl[c:c + 1, :]
        d = d + diff * diff
    d = jnp.where(i0 >= _N, _INF, d)
    sels = []
    for j in range(_K4):
        mn = jnp.min(d, axis=0, keepdims=True)
        sel = jnp.min(jnp.where(d == mn, i0, _Q), axis=0, keepdims=True)
        sels.append(sel)
        d = jnp.where(i0 == sel, _INF, d)
    idx_pp[...] = jnp.concatenate(sels, axis=0)


def _stage_a(*args):
    f32, i32 = jnp.float32, jnp.int32
    outs = (
        jax.ShapeDtypeStruct((2, _Q, _CF), f32),
        jax.ShapeDtypeStruct((2, _Q, _CF), f32),
        jax.ShapeDtypeStruct((_Q, _NP), i32),
        jax.ShapeDtypeStruct((_Q, 16), f32),
        jax.ShapeDtypeStruct((_Q, _NP), i32),
        jax.ShapeDtypeStruct((_Q, 16), f32),
        jax.ShapeDtypeStruct((_K4, _Q), i32),
    )
    return pl.pallas_call(_stage_a_body, out_shape=outs)(*args)


_QS = _Q // 16
_GS = _QS * _NP


def _sc_pool():
    f32, i32 = jnp.float32, jnp.int32
    mesh = plsc.VectorSubcoreMesh(core_axis_name="c", subcore_axis_name="s",
                                  num_cores=2, num_subcores=16)
    scratch = [
        pltpu.VMEM((4, _GH), i32),
        pltpu.VMEM((_QS * 16,), f32),
        pltpu.VMEM((4, _GH), i32),
        pltpu.VMEM((_QS * 16,), f32),
        pltpu.VMEM((_GS, _CF), f32),
        pltpu.VMEM((_QS, _CF), f32),
        pltpu.VMEM((_QS, _CF), f32),
        pltpu.VMEM((_QS, _CF), f32),
        pltpu.VMEM_SHARED((_Q, _CF), f32),
        pltpu.SemaphoreType.DMA,
        pltpu.SemaphoreType.DMA,
    ]
    out_type = (jax.ShapeDtypeStruct((2, _Q, _CF), f32),
                jax.ShapeDtypeStruct((2, _Q, _CF), f32))

    @functools.partial(pl.kernel, out_type=out_type, mesh=mesh,
                       scratch_types=scratch)
    def k(idxa_hbm, wa_hbm, idxb_hbm, wb_hbm, tabG, ownG,
          diffG, sG, idx_v, w_v, idx2_v, w2_v, rows, own_v, oD, oS, spm,
          sem, sem2):
        c = lax.axis_index("c")
        s = lax.axis_index("s")
        bq = s * _QS

        def pool(oref, sub, wv):
            @plsc.parallel_loop(0, _QS, 1, unroll=2)
            def body(q):
                b0 = q * _NP
                wq = wv[pl.ds(q * 16, 16)]
                for ch in range(_CF // 16):
                    sl = pl.ds(ch * 16, 16)
                    m = rows[b0, sl] * wq[0]
                    for j in range(1, _NP):
                        m = jnp.maximum(m, rows[b0 + j, sl] * wq[j])
                    if sub:
                        oref[q, sl] = own_v[q, sl] - m
                    else:
                        oref[q, sl] = m

        pltpu.sync_copy(idxa_hbm.at[pl.ds(s * 4, 4)], idx_v)
        pltpu.sync_copy(wa_hbm.at[pl.ds(s * _QS * 16, _QS * 16)], w_v)
        cps = [pltpu.async_copy(tabG.at[c].at[idx_v.at[h]],
                                rows.at[pl.ds(h * _GH, _GH)], sem)
               for h in range(4)]
        pltpu.sync_copy(ownG.at[c].at[pl.ds(bq, _QS)], own_v)
        pltpu.sync_copy(idxb_hbm.at[pl.ds(s * 4, 4)], idx2_v)
        pltpu.sync_copy(wb_hbm.at[pl.ds(s * _QS * 16, _QS * 16)], w2_v)
        for cp in cps:
            cp.wait()
        pool(oD, True, w_v)
        pltpu.sync_copy(oD, spm.at[pl.ds(bq, _QS)])
        hbm_cp = pltpu.async_copy(oD, diffG.at[c].at[pl.ds(bq, _QS)], sem2)
        plsc.subcore_barrier()

        cps = [pltpu.async_copy(spm.at[idx2_v.at[h]],
                                rows.at[pl.ds(h * _GH, _GH)], sem)
               for h in range(4)]
        for cp in cps:
            cp.wait()
        pool(oS, False, w2_v)
        pltpu.sync_copy(oS, sG.at[c].at[pl.ds(bq, _QS)])
        hbm_cp.wait()

    return k


def _stage_e_body(diffG, sG, tgt_r, cur_r, ipp,
                  wfc1, bfc1, wfc2, bfc2, wfu, bfu,
                  wp1, bp1, wp2, bp2, wp3, bp3, out):
    fi = _mm(jnp.concatenate([diffG[0], sG[1]], axis=1), wfc1[...]) + bfc1[...]
    fp = _mm(jnp.concatenate([diffG[1], sG[0]], axis=1), wfc2[...]) + bfc2[...]
    ft = _mm(jnp.concatenate([fp, fi], axis=1), wfu[...]) + bfu[...]
    x = _mm(ft, wp1[...]) + bp1[...]
    x = _lrelu(_mm(x, wp2[...]) + bp2[...])
    x = _mm(x, wp3[...]) + bp3[...]
    xT = jnp.transpose(x)
    tf = jnp.concatenate(
        [tgt_r[0] * xT[:, :_N], jnp.zeros((160, _Q - _N), jnp.float32)],
        axis=1)
    ii = lax.broadcasted_iota(jnp.int32, (_Q, _Q), 0)
    ippv = ipp[...]
    s = jnp.zeros((_Q, _Q), jnp.float32)
    for j in range(_K4):
        s = s + jnp.where(ii == ippv[j:j + 1, :], 1.0, 0.0)
    g = lax.dot_general(tf, s, (((1,), (0,)), ((), ())),
                        preferred_element_type=jnp.float32)
    out[...] = cur_r[0] + 0.25 * g[:, :_N]


def _stage_e(*args):
    return pl.pallas_call(
        _stage_e_body,
        out_shape=jax.ShapeDtypeStruct((160, _N), jnp.float32))(*args)


def kernel(img, cloud, img_tar, cloud_tar, current_feat, target_feat,
           w_conv1, b_conv1, w_conv2, b_conv2, w_pconv1, b_pconv1,
           w_pconv2, b_pconv2, w_fc1, b_fc1, w_fc2, b_fc2,
           w_fuse2, b_fuse2, w_pn1, b_pn1, w_pn2, b_pn2, w_pn3, b_pn3):
    row = lambda b: b[None, :]

    (fG, ownG, idx_ab, w_ab, idx_bb, w_bb, idx_pp) = _stage_a(
        img, cloud, img_tar, cloud_tar,
        w_conv1, row(b_conv1), w_conv2, row(b_conv2),
        w_pconv1, row(b_pconv1), w_pconv2, row(b_pconv2))

    idx_ab_f = idx_ab.reshape(-1, _GH)
    idx_bb_f = idx_bb.reshape(-1, _GH)
    w_ab_f = w_ab.reshape(-1)
    w_bb_f = w_bb.reshape(-1)

    diffG, sG = _sc_pool()(idx_ab_f, w_ab_f, idx_bb_f, w_bb_f, fG, ownG)

    outp = _stage_e(diffG, sG, target_feat, current_feat, idx_pp,
                    w_fc1, row(b_fc1), w_fc2, row(b_fc2),
                    w_fuse2, row(b_fuse2), w_pn1, row(b_pn1),
                    w_pn2, row(b_pn2), w_pn3, row(b_pn3))
    return outp[None]
